# baseline (speedup 1.0000x reference)
import jax
import jax.numpy as jnp
from jax.experimental import pallas as pl
from jax.experimental.pallas import tpu as pltpu

_N_BEHAVIORS = 8
_LB = 8
_NB = 2


def _body(pb_ref, thr_ref, beh_ref, pad_ref, s_ref, hp_ref, hn_ref):
    beh = beh_ref[...]
    pad = pad_ref[...]
    i = pl.program_id(0)
    thr = thr_ref[pl.ds(i * _LB, _LB), :]
    t = 1.0 / (1.0 + jnp.exp(-thr))
    idx = jnp.maximum(beh - 1, 0)
    pref = jnp.zeros_like(pad)
    for k in range(_N_BEHAVIORS):
        pref = jnp.where(idx == k, pb_ref[0, k], pref)
    signal = pref - t
    m = (signal > 0).astype(jnp.float32) * pad
    hnf = (1.0 - m) * pad
    s = s_ref[...]
    hp_ref[...] = s * m[:, None, :]
    hn_ref[...] = s * hnf[:, None, :]


def kernel(S, behavior_seq, padding_mask, lambda_raw, threshold):
    B, L, D = S.shape
    lam = jax.nn.softplus(lambda_raw) + 1e-6
    log_pmf = -lam + lam * jnp.log(lam) - jax.lax.lgamma(lam + 1.0)
    p_b = (jnp.exp(log_pmf) + 1.0).reshape(1, _N_BEHAVIORS)
    thr2 = threshold[:L].reshape(L, 1)
    s_t = jnp.transpose(S, (1, 2, 0))
    beh_t = behavior_seq.T
    pad_t = padding_mask.T

    BB = B // _NB
    grid = (L // _LB, _NB)
    out = pl.pallas_call(
        _body,
        grid=grid,
        in_specs=[
            pl.BlockSpec(memory_space=pltpu.SMEM),
            pl.BlockSpec((L, 1), lambda i, j: (0, 0)),
            pl.BlockSpec((_LB, BB), lambda i, j: (i, j)),
            pl.BlockSpec((_LB, BB), lambda i, j: (i, j)),
            pl.BlockSpec((_LB, D, BB), lambda i, j: (i, 0, j)),
        ],
        out_specs=[
            pl.BlockSpec((_LB, D, BB), lambda i, j: (i, 0, j)),
            pl.BlockSpec((_LB, D, BB), lambda i, j: (i, 0, j)),
        ],
        out_shape=[
            jax.ShapeDtypeStruct((L, D, B), jnp.float32),
            jax.ShapeDtypeStruct((L, D, B), jnp.float32),
        ],
    )(p_b, thr2, beh_t, pad_t, s_t)
    return (jnp.transpose(out[0], (2, 0, 1)), jnp.transpose(out[1], (2, 0, 1)))
